# baseline (speedup 1.0000x reference)
import functools

import jax
import jax.numpy as jnp
from jax import lax
from jax.experimental import pallas as pl
from jax.experimental.pallas import tpu as pltpu
from jax.experimental.pallas import tpu_sc as plsc

NC = 2
NS = 16
NW = NC * NS
CH = 128
G = 64


def _mesh():
    return plsc.VectorSubcoreMesh(core_axis_name="c", subcore_axis_name="s")


_GDN = jax.lax.GatherDimensionNumbers(
    offset_dims=(), collapsed_slice_dims=(0,), start_index_map=(0,))


def _splat(vec16, lane):
    idx = jnp.full((16,), lane, jnp.int32)[:, None]
    return jax.lax.gather(vec16, idx, _GDN, (1,),
                          mode=jax.lax.GatherScatterMode.PROMISE_IN_BOUNDS)


def _slice_split(n, nw=NS):
    per = ((n + nw - 1) // nw + 7) // 8 * 8
    out = []
    for t in range(nw):
        lo = t * per
        hi = min(n, lo + per)
        out.append((lo, max(0, hi - lo)))
    return out


def _make_deg(N, n_chunks_per_tile):
    ranges = _slice_split(N)
    cpt = n_chunks_per_tile

    @functools.partial(
        pl.kernel, mesh=_mesh(),
        out_type=jax.ShapeDtypeStruct((NC * N,), jnp.float32),
        scratch_types=[
            pltpu.VMEM((2, CH), jnp.int32),
            pltpu.VMEM((2, CH), jnp.int32),
            pltpu.VMEM((CH,), jnp.float32),
            pltpu.VMEM((CH,), jnp.float32),
            pltpu.VMEM((640,), jnp.float32),
            pltpu.VMEM_SHARED((N,), jnp.float32),
            pltpu.SemaphoreType.DMA,
            pltpu.SemaphoreType.DMA,
            pltpu.SemaphoreType.DMA,
            pltpu.SemaphoreType.DMA,
        ],
    )
    def deg_k(eidx_hbm, eww_hbm, out_hbm, eb0, eb1, ewb0, ewb1, zbuf, acc,
              cs0, cs1, ss0, ss1):
        c = lax.axis_index("c")
        s = lax.axis_index("s")
        wid = c * NS + s
        ebs, ewbs = (eb0, eb1), (ewb0, ewb1)
        csems, ssems = (cs0, cs1), (ss0, ss1)

        def issue(p, k):
            pltpu.async_copy(eidx_hbm.at[k * NW + wid], ebs[p], csems[p])
            pltpu.async_copy(eww_hbm.at[k * NW + wid], ewbs[p], csems[p])

        def wait_issue(p, k):
            pltpu.make_async_copy(eidx_hbm.at[k * NW + wid], ebs[p],
                                  csems[p]).wait()
            pltpu.make_async_copy(eww_hbm.at[k * NW + wid], ewbs[p],
                                  csems[p]).wait()

        def start_scatter(p):
            pltpu.async_copy(ewbs[p], acc.at[ebs[p].at[1]], ssems[p],
                             add=True)

        def wait_scatter(p):
            pltpu.make_async_copy(ewbs[p], acc.at[ebs[p].at[1]],
                                  ssems[p]).wait()

        issue(0, 0)
        issue(1, 1)

        def zb(i, _):
            zbuf[pl.ds(i * 16, 16)] = jnp.zeros((16,), jnp.float32)
            return _
        lax.fori_loop(0, 640 // 16, zb, None)

        for t, (lo, ln) in enumerate(ranges):
            if ln == 0:
                continue
            @pl.when(s == t)
            def _():
                pltpu.sync_copy(zbuf.at[pl.ds(0, ln)],
                                acc.at[pl.ds(lo, ln)])
        plsc.subcore_barrier()

        def body(k, _):
            b = 0
            wait_issue(b, k)
            start_scatter(b)
            wait_scatter(b)

            @pl.when(k + 2 < cpt)
            def _():
                issue(b, k + 2)
            return _

        def body1(k, _):
            b = 1
            wait_issue(b, k)
            start_scatter(b)
            wait_scatter(b)

            @pl.when(k + 2 < cpt)
            def _():
                issue(b, k + 2)
            return _

        def both(i, _):
            body(2 * i, None)
            body1(2 * i + 1, None)
            return _
        lax.fori_loop(0, cpt // 2, both, None)
        if cpt % 2:
            body(cpt - 1, None)
        plsc.subcore_barrier()

        for t, (lo, ln) in enumerate(ranges):
            if ln == 0:
                continue
            @pl.when(s == t)
            def _():
                pltpu.sync_copy(acc.at[pl.ds(lo, ln)], zbuf.at[pl.ds(0, ln)])
                pltpu.sync_copy(zbuf.at[pl.ds(0, ln)],
                                out_hbm.at[pl.ds(c * N + lo, ln)])

    return deg_k


NB = 3


def _make_agg(N, D, n_chunks_per_tile):
    ranges = _slice_split(N)
    nf = D // 16
    cpt = n_chunks_per_tile
    assert cpt % NB == 0

    @functools.partial(
        pl.kernel, mesh=_mesh(),
        out_type=jax.ShapeDtypeStruct((NC, N, D), jnp.float32),
        scratch_types=(
            [pltpu.VMEM((2, CH), jnp.int32) for _ in range(NB)]
            + [pltpu.VMEM((CH,), jnp.float32) for _ in range(NB)]
            + [pltpu.VMEM((CH, D), jnp.float32) for _ in range(NB)]
            + [pltpu.VMEM_SHARED((N, D), jnp.float32)]
            + [pltpu.SemaphoreType.DMA for _ in range(2 * NB)]
        ),
    )
    def agg_k(eidx_hbm, eww_hbm, h_hbm, out_hbm, *bufs):
        ebs = bufs[0:NB]
        ewbs = bufs[NB:2 * NB]
        rws = bufs[2 * NB:3 * NB]
        acc = bufs[3 * NB]
        gsems = bufs[3 * NB + 1:3 * NB + 1 + NB]
        ssems = bufs[3 * NB + 1 + NB:3 * NB + 1 + 2 * NB]
        zbuf = rws[0]
        c = lax.axis_index("c")
        s = lax.axis_index("s")
        wid = c * NS + s

        def issue(p, k):
            pltpu.sync_copy(eidx_hbm.at[k * NW + wid], ebs[p])
            pltpu.sync_copy(eww_hbm.at[k * NW + wid], ewbs[p])
            pltpu.async_copy(h_hbm.at[ebs[p].at[0]], rws[p], gsems[p])

        def wait_gather(p):
            pltpu.make_async_copy(h_hbm.at[ebs[p].at[0]], rws[p],
                                  gsems[p]).wait()

        def start_scatter(p):
            pltpu.async_copy(rws[p], acc.at[ebs[p].at[1]], ssems[p], add=True)

        def wait_scatter(p):
            pltpu.make_async_copy(rws[p], acc.at[ebs[p].at[1]],
                                  ssems[p]).wait()

        def scale(p):
            rows, ewb = rws[p], ewbs[p]

            def grp(g, _):
                ev = ewb[pl.ds(pl.multiple_of(g * 16, 16), 16)]
                for jj in range(16):
                    sp = _splat(ev, jj)
                    r = g * 16 + jj
                    for f in range(nf):
                        sl = pl.ds(f * 16, 16)
                        rows[r, sl] = rows[r, sl] * sp
                return _
            lax.fori_loop(0, CH // 16, grp, None)

        def zb(i, _):
            for f in range(nf):
                zbuf[i, pl.ds(f * 16, 16)] = jnp.zeros((16,), jnp.float32)
            return _
        lax.fori_loop(0, CH, zb, None)

        for t, (lo, ln) in enumerate(ranges):
            if ln == 0:
                continue
            @pl.when(s == t)
            def _():
                for q0 in range(0, ln, CH):
                    w = min(CH, ln - q0)
                    pltpu.sync_copy(zbuf.at[pl.ds(0, w)],
                                    acc.at[pl.ds(lo + q0, w)])
        for b in range(NB):
            issue(b, b)
        plsc.subcore_barrier()

        def body(i, _):
            k = i * NB
            for b in range(NB):
                wait_gather(b)
                scale(b)
                start_scatter(b)
                wait_scatter(b)

                @pl.when(k + b + NB < cpt)
                def _():
                    issue(b, k + b + NB)
            return _
        lax.fori_loop(0, cpt // NB, body, None)
        plsc.subcore_barrier()

        for t, (lo, ln) in enumerate(ranges):
            if ln == 0:
                continue
            @pl.when(s == t)
            def _():
                for q0 in range(0, ln, CH):
                    w = min(CH, ln - q0)
                    pltpu.sync_copy(acc.at[pl.ds(lo + q0, w)],
                                    zbuf.at[pl.ds(0, w)])
                    pltpu.sync_copy(zbuf.at[pl.ds(0, w)],
                                    out_hbm.at[c, pl.ds(lo + q0, w)])

    return agg_k


def _stage_a(degp, x, W1, R):
    N, D = x.shape
    H = W1.shape[1]
    grid = (N // R,)

    def body(p_ref, x_ref, w_ref, h_ref, dis_ref):
        deg = p_ref[:, 0:1] + p_ref[:, 1:2] + 1.0
        dis = jnp.where(deg > 0, jax.lax.rsqrt(deg), 0.0)
        h = jnp.dot(x_ref[...], w_ref[...], preferred_element_type=jnp.float32)
        h_ref[...] = h * dis
        dis_ref[...] = dis

    return pl.pallas_call(
        body,
        grid=grid,
        in_specs=[
            pl.BlockSpec((R, NC), lambda i: (i, 0)),
            pl.BlockSpec((R, D), lambda i: (i, 0)),
            pl.BlockSpec((D, H), lambda i: (0, 0)),
        ],
        out_specs=[
            pl.BlockSpec((R, H), lambda i: (i, 0)),
            pl.BlockSpec((R, 1), lambda i: (i, 0)),
        ],
        out_shape=[
            jax.ShapeDtypeStruct((N, H), jnp.float32),
            jax.ShapeDtypeStruct((N, 1), jnp.float32),
        ],
    )(degp, x, W1)


def _stage_b(agg, h1p, dis, b1, W2, R):
    N, H = h1p.shape
    H2 = W2.shape[1]
    grid = (N // R,)

    def body(a_ref, h_ref, d_ref, b_ref, w_ref, o_ref):
        dis = d_ref[...]
        pre = (a_ref[0] + a_ref[1] + h_ref[...]) * dis + b_ref[...]
        out1 = jnp.maximum(pre, 0.0)
        h2 = jnp.dot(out1, w_ref[...], preferred_element_type=jnp.float32)
        o_ref[...] = h2 * dis

    return pl.pallas_call(
        body,
        grid=grid,
        in_specs=[
            pl.BlockSpec((NC, R, H), lambda i: (0, i, 0)),
            pl.BlockSpec((R, H), lambda i: (i, 0)),
            pl.BlockSpec((R, 1), lambda i: (i, 0)),
            pl.BlockSpec((1, H), lambda i: (0, 0)),
            pl.BlockSpec((H, H2), lambda i: (0, 0)),
        ],
        out_specs=pl.BlockSpec((R, H2), lambda i: (i, 0)),
        out_shape=jax.ShapeDtypeStruct((N, H2), jnp.float32),
    )(agg, h1p, dis, b1, W2)


def _stage_c(agg2, h2p, dis, b2, batch, fW1, fb1, fW2p, fb2p, HV, R):
    N, H2 = h2p.shape
    FH = fW1.shape[1]
    grid = (N // R,)
    nsteps = grid[0]

    def body(a_ref, h_ref, d_ref, b_ref, bat_ref, w1_ref, c1_ref,
             w2_ref, c2_ref, o_ref, acc_ref):
        i = pl.program_id(0)
        dis = d_ref[...]
        pre = (a_ref[0] + a_ref[1] + h_ref[...]) * dis + b_ref[...]
        x1 = jnp.maximum(pre, 0.0)
        ccol = jax.lax.broadcasted_iota(jnp.int32, (R, H2), 1)
        aug = jnp.where(ccol == HV, 1.0, x1)
        gid = jax.lax.broadcasted_iota(jnp.int32, (G, R), 0)
        oh = (bat_ref[...][:, 0][None, :] == gid).astype(jnp.float32)
        contrib = jnp.dot(oh, aug, preferred_element_type=jnp.float32)

        @pl.when(i == 0)
        def _():
            acc_ref[...] = jnp.zeros_like(acc_ref)
        acc_ref[...] += contrib

        @pl.when(i == nsteps - 1)
        def _():
            acc = acc_ref[...]
            seg = acc[:, :HV]
            cnt = acc[:, HV][:, None]
            x2 = seg / jnp.maximum(cnt, 1.0)
            hfc = jnp.maximum(
                jnp.dot(x2, w1_ref[...], preferred_element_type=jnp.float32)
                + c1_ref[...], 0.0)
            logits = jnp.dot(hfc, w2_ref[...],
                             preferred_element_type=jnp.float32) + c2_ref[...]
            col = jax.lax.broadcasted_iota(jnp.int32, (G, 128), 1)
            valid = col < 2
            neg = jnp.full_like(logits, -jnp.inf)
            m = jnp.max(jnp.where(valid, logits, neg), axis=1, keepdims=True)
            e = jnp.where(valid, jnp.exp(logits - m), 0.0)
            lse = jnp.log(jnp.sum(e, axis=1, keepdims=True))
            res = logits - m - lse
            o_ref[...] = res[:, :2]

    return pl.pallas_call(
        body,
        grid=grid,
        in_specs=[
            pl.BlockSpec((NC, R, H2), lambda i: (0, i, 0)),
            pl.BlockSpec((R, H2), lambda i: (i, 0)),
            pl.BlockSpec((R, 1), lambda i: (i, 0)),
            pl.BlockSpec((1, H2), lambda i: (0, 0)),
            pl.BlockSpec((R, 1), lambda i: (i, 0)),
            pl.BlockSpec((HV, FH), lambda i: (0, 0)),
            pl.BlockSpec((1, FH), lambda i: (0, 0)),
            pl.BlockSpec((FH, 128), lambda i: (0, 0)),
            pl.BlockSpec((1, 128), lambda i: (0, 0)),
        ],
        out_specs=pl.BlockSpec((G, 2), lambda i: (0, 0)),
        out_shape=jax.ShapeDtypeStruct((G, 2), jnp.float32),
        scratch_shapes=[pltpu.VMEM((G, 128), jnp.float32)],
    )(agg2, h2p, dis, b2, batch, fW1, fb1, fW2p, fb2p)


def kernel(x, edge_index, edge_attr, batch, W1, b1, W2, b2,
           fW1, fb1, fW2, fb2):
    N, D = x.shape
    H1 = W1.shape[1]
    H2 = W2.shape[1]
    E = edge_index.shape[1]

    src = edge_index[0]
    dst = edge_index[1]
    ew = edge_attr[:, 0]

    per = NW * CH * 3
    Epad = (E + per - 1) // per * per
    pad = Epad - E
    if pad:
        zi = jnp.zeros((pad,), jnp.int32)
        src = jnp.concatenate([src, zi])
        dst = jnp.concatenate([dst, zi])
        ew = jnp.concatenate([ew, jnp.zeros((pad,), jnp.float32)])
    cpt = Epad // (NW * CH)
    nchunks = Epad // CH
    eidx = jnp.stack([src, dst], axis=0)
    eidx = eidx.reshape(2, nchunks, CH).transpose(1, 0, 2)
    eww = ew.reshape(nchunks, CH)

    degp = _make_deg(N, cpt)(eidx, eww).reshape(NC, N)
    degp_t = degp.T

    R = 1000
    h1p, dis = _stage_a(degp_t, x, W1, R)
    agg1 = _make_agg(N, H1, cpt)(eidx, eww, h1p)
    W2p = jnp.pad(W2, ((0, 0), (0, 128 - H2)))
    h2p = _stage_b(agg1, h1p, dis, b1.reshape(1, -1), W2p, R)
    agg2 = _make_agg(N, 128, cpt)(eidx, eww, h2p)

    b2p = jnp.pad(b2, (0, 128 - H2)).reshape(1, -1)
    fW2p = jnp.pad(fW2, ((0, 0), (0, 128 - fW2.shape[1])))
    fb2p = jnp.pad(fb2, (0, 128 - fb2.shape[0])).reshape(1, -1)
    out = _stage_c(agg2, h2p, dis, b2p, batch.reshape(-1, 1),
                   fW1, fb1.reshape(1, -1), fW2p, fb2p, H2, R)
    return out


# speedup vs baseline: 1.3081x; 1.3081x over previous
import functools

import jax
import jax.numpy as jnp
from jax import lax
from jax.experimental import pallas as pl
from jax.experimental.pallas import tpu as pltpu
from jax.experimental.pallas import tpu_sc as plsc

NC = 2
NS = 16
NW = NC * NS
CH = 128
G = 64


def _mesh():
    return plsc.VectorSubcoreMesh(core_axis_name="c", subcore_axis_name="s")


_GDN = jax.lax.GatherDimensionNumbers(
    offset_dims=(), collapsed_slice_dims=(0,), start_index_map=(0,))


def _splat(vec16, lane):
    idx = jnp.full((16,), lane, jnp.int32)[:, None]
    return jax.lax.gather(vec16, idx, _GDN, (1,),
                          mode=jax.lax.GatherScatterMode.PROMISE_IN_BOUNDS)


def _slice_split(n, nw=NS):
    per = ((n + nw - 1) // nw + 7) // 8 * 8
    out = []
    for t in range(nw):
        lo = t * per
        hi = min(n, lo + per)
        out.append((lo, max(0, hi - lo)))
    return out


def _make_deg(N, n_chunks_per_tile):
    ranges = _slice_split(N)
    cpt = n_chunks_per_tile

    @functools.partial(
        pl.kernel, mesh=_mesh(),
        out_type=jax.ShapeDtypeStruct((NC * N,), jnp.float32),
        scratch_types=[
            pltpu.VMEM((2, CH), jnp.int32),
            pltpu.VMEM((2, CH), jnp.int32),
            pltpu.VMEM((CH,), jnp.float32),
            pltpu.VMEM((CH,), jnp.float32),
            pltpu.VMEM((640,), jnp.float32),
            pltpu.VMEM_SHARED((N,), jnp.float32),
            pltpu.SemaphoreType.DMA,
            pltpu.SemaphoreType.DMA,
            pltpu.SemaphoreType.DMA,
            pltpu.SemaphoreType.DMA,
        ],
    )
    def deg_k(eidx_hbm, eww_hbm, out_hbm, eb0, eb1, ewb0, ewb1, zbuf, acc,
              cs0, cs1, ss0, ss1):
        c = lax.axis_index("c")
        s = lax.axis_index("s")
        wid = c * NS + s
        ebs, ewbs = (eb0, eb1), (ewb0, ewb1)
        csems, ssems = (cs0, cs1), (ss0, ss1)

        def issue(p, k):
            pltpu.async_copy(eidx_hbm.at[k * NW + wid], ebs[p], csems[p])
            pltpu.async_copy(eww_hbm.at[k * NW + wid], ewbs[p], csems[p])

        def wait_issue(p, k):
            pltpu.make_async_copy(eidx_hbm.at[k * NW + wid], ebs[p],
                                  csems[p]).wait()
            pltpu.make_async_copy(eww_hbm.at[k * NW + wid], ewbs[p],
                                  csems[p]).wait()

        def start_scatter(p):
            pltpu.async_copy(ewbs[p], acc.at[ebs[p].at[1]], ssems[p],
                             add=True)

        def wait_scatter(p):
            pltpu.make_async_copy(ewbs[p], acc.at[ebs[p].at[1]],
                                  ssems[p]).wait()

        issue(0, 0)
        issue(1, 1)

        def zb(i, _):
            zbuf[pl.ds(i * 16, 16)] = jnp.zeros((16,), jnp.float32)
            return _
        lax.fori_loop(0, 640 // 16, zb, None)

        for t, (lo, ln) in enumerate(ranges):
            if ln == 0:
                continue
            @pl.when(s == t)
            def _():
                pltpu.sync_copy(zbuf.at[pl.ds(0, ln)],
                                acc.at[pl.ds(lo, ln)])
        plsc.subcore_barrier()

        def body(k, _):
            b = 0
            wait_issue(b, k)
            start_scatter(b)
            wait_scatter(b)

            @pl.when(k + 2 < cpt)
            def _():
                issue(b, k + 2)
            return _

        def body1(k, _):
            b = 1
            wait_issue(b, k)
            start_scatter(b)
            wait_scatter(b)

            @pl.when(k + 2 < cpt)
            def _():
                issue(b, k + 2)
            return _

        def both(i, _):
            body(2 * i, None)
            body1(2 * i + 1, None)
            return _
        lax.fori_loop(0, cpt // 2, both, None)
        if cpt % 2:
            body(cpt - 1, None)
        plsc.subcore_barrier()

        for t, (lo, ln) in enumerate(ranges):
            if ln == 0:
                continue
            @pl.when(s == t)
            def _():
                pltpu.sync_copy(acc.at[pl.ds(lo, ln)], zbuf.at[pl.ds(0, ln)])
                pltpu.sync_copy(zbuf.at[pl.ds(0, ln)],
                                out_hbm.at[pl.ds(c * N + lo, ln)])

    return deg_k


NB = 2


def _make_agg(N, D, n_chunks_per_tile):
    ranges = _slice_split(N)
    nf = D // 16
    cpt = n_chunks_per_tile
    assert cpt % NB == 0

    @functools.partial(
        pl.kernel, mesh=_mesh(),
        out_type=jax.ShapeDtypeStruct((NC, N, D), jnp.float32),
        scratch_types=(
            [pltpu.VMEM((2, CH), jnp.int32) for _ in range(NB)]
            + [pltpu.VMEM((CH,), jnp.float32) for _ in range(NB)]
            + [pltpu.VMEM((CH, D), jnp.float32) for _ in range(NB)]
            + [pltpu.VMEM_SHARED((N, D), jnp.float32)]
            + [pltpu.SemaphoreType.DMA for _ in range(2 * NB)]
        ),
    )
    def agg_k(eidx_hbm, eww_hbm, h_hbm, out_hbm, *bufs):
        ebs = bufs[0:NB]
        ewbs = bufs[NB:2 * NB]
        rws = bufs[2 * NB:3 * NB]
        acc = bufs[3 * NB]
        gsems = bufs[3 * NB + 1:3 * NB + 1 + NB]
        ssems = bufs[3 * NB + 1 + NB:3 * NB + 1 + 2 * NB]
        zbuf = rws[0]
        c = lax.axis_index("c")
        s = lax.axis_index("s")
        wid = c * NS + s

        def issue(p, k):
            pltpu.sync_copy(eidx_hbm.at[k * NW + wid], ebs[p])
            pltpu.sync_copy(eww_hbm.at[k * NW + wid], ewbs[p])
            pltpu.async_copy(h_hbm.at[ebs[p].at[0]], rws[p], gsems[p])

        def wait_gather(p):
            pltpu.make_async_copy(h_hbm.at[ebs[p].at[0]], rws[p],
                                  gsems[p]).wait()

        def start_scatter(p):
            pltpu.async_copy(rws[p], acc.at[ebs[p].at[1]], ssems[p], add=True)

        def wait_scatter(p):
            pltpu.make_async_copy(rws[p], acc.at[ebs[p].at[1]],
                                  ssems[p]).wait()

        def scale(p):
            rows, ewb = rws[p], ewbs[p]

            def grp(g, _):
                ev = ewb[pl.ds(pl.multiple_of(g * 16, 16), 16)]
                for jj in range(16):
                    sp = _splat(ev, jj)
                    r = g * 16 + jj
                    for f in range(nf):
                        sl = pl.ds(f * 16, 16)
                        rows[r, sl] = rows[r, sl] * sp
                return _
            lax.fori_loop(0, CH // 16, grp, None)

        def zb(i, _):
            for f in range(nf):
                zbuf[i, pl.ds(f * 16, 16)] = jnp.zeros((16,), jnp.float32)
            return _
        lax.fori_loop(0, CH, zb, None)

        for t, (lo, ln) in enumerate(ranges):
            if ln == 0:
                continue
            @pl.when(s == t)
            def _():
                for q0 in range(0, ln, CH):
                    w = min(CH, ln - q0)
                    pltpu.sync_copy(zbuf.at[pl.ds(0, w)],
                                    acc.at[pl.ds(lo + q0, w)])
        for b in range(NB):
            issue(b, b)
        plsc.subcore_barrier()

        def body(i, _):
            k = i * NB
            wait_gather(0)
            scale(0)
            start_scatter(0)
            wait_gather(1)
            scale(1)
            start_scatter(1)

            @pl.when(k + NB < cpt)
            def _():
                wait_scatter(0)
                issue(0, k + NB)
                wait_scatter(1)
                issue(1, k + NB + 1)
            return _
        lax.fori_loop(0, cpt // NB, body, None)
        wait_scatter(0)
        wait_scatter(1)
        plsc.subcore_barrier()

        for t, (lo, ln) in enumerate(ranges):
            if ln == 0:
                continue
            @pl.when(s == t)
            def _():
                for q0 in range(0, ln, CH):
                    w = min(CH, ln - q0)
                    pltpu.sync_copy(acc.at[pl.ds(lo + q0, w)],
                                    zbuf.at[pl.ds(0, w)])
                    pltpu.sync_copy(zbuf.at[pl.ds(0, w)],
                                    out_hbm.at[c, pl.ds(lo + q0, w)])

    return agg_k


def _stage_a1(x, W1, R):
    N, D = x.shape
    H = W1.shape[1]
    grid = (N // R,)

    def body(x_ref, w_ref, o_ref):
        o_ref[...] = jnp.dot(x_ref[...], w_ref[...],
                             preferred_element_type=jnp.float32)

    return pl.pallas_call(
        body,
        grid=grid,
        in_specs=[
            pl.BlockSpec((R, D), lambda i: (i, 0)),
            pl.BlockSpec((D, H), lambda i: (0, 0)),
        ],
        out_specs=pl.BlockSpec((R, H), lambda i: (i, 0)),
        out_shape=jax.ShapeDtypeStruct((N, H), jnp.float32),
    )(x, W1)


def _stage_a2(degp, xw, R):
    N, H = xw.shape
    grid = (N // R,)

    def body(p_ref, xw_ref, h_ref, dis_ref):
        deg = p_ref[:, 0:1] + p_ref[:, 1:2] + 1.0
        dis = jnp.where(deg > 0, jax.lax.rsqrt(deg), 0.0)
        h_ref[...] = xw_ref[...] * dis
        dis_ref[...] = dis

    return pl.pallas_call(
        body,
        grid=grid,
        in_specs=[
            pl.BlockSpec((R, NC), lambda i: (i, 0)),
            pl.BlockSpec((R, H), lambda i: (i, 0)),
        ],
        out_specs=[
            pl.BlockSpec((R, H), lambda i: (i, 0)),
            pl.BlockSpec((R, 1), lambda i: (i, 0)),
        ],
        out_shape=[
            jax.ShapeDtypeStruct((N, H), jnp.float32),
            jax.ShapeDtypeStruct((N, 1), jnp.float32),
        ],
    )(degp, xw)


def _stage_b(agg, h1p, dis, b1, W2, R):
    N, H = h1p.shape
    H2 = W2.shape[1]
    grid = (N // R,)

    def body(a_ref, h_ref, d_ref, b_ref, w_ref, o_ref):
        dis = d_ref[...]
        pre = (a_ref[0] + a_ref[1] + h_ref[...]) * dis + b_ref[...]
        out1 = jnp.maximum(pre, 0.0)
        h2 = jnp.dot(out1, w_ref[...], preferred_element_type=jnp.float32)
        o_ref[...] = h2 * dis

    return pl.pallas_call(
        body,
        grid=grid,
        in_specs=[
            pl.BlockSpec((NC, R, H), lambda i: (0, i, 0)),
            pl.BlockSpec((R, H), lambda i: (i, 0)),
            pl.BlockSpec((R, 1), lambda i: (i, 0)),
            pl.BlockSpec((1, H), lambda i: (0, 0)),
            pl.BlockSpec((H, H2), lambda i: (0, 0)),
        ],
        out_specs=pl.BlockSpec((R, H2), lambda i: (i, 0)),
        out_shape=jax.ShapeDtypeStruct((N, H2), jnp.float32),
    )(agg, h1p, dis, b1, W2)


def _stage_c(agg2, h2p, dis, b2, batch, fW1, fb1, fW2p, fb2p, HV, R):
    N, H2 = h2p.shape
    FH = fW1.shape[1]
    grid = (N // R,)
    nsteps = grid[0]

    def body(a_ref, h_ref, d_ref, b_ref, bat_ref, w1_ref, c1_ref,
             w2_ref, c2_ref, o_ref, acc_ref):
        i = pl.program_id(0)
        dis = d_ref[...]
        pre = (a_ref[0] + a_ref[1] + h_ref[...]) * dis + b_ref[...]
        x1 = jnp.maximum(pre, 0.0)
        ccol = jax.lax.broadcasted_iota(jnp.int32, (R, H2), 1)
        aug = jnp.where(ccol == HV, 1.0, x1)
        gid = jax.lax.broadcasted_iota(jnp.int32, (G, R), 0)
        oh = (bat_ref[...][:, 0][None, :] == gid).astype(jnp.float32)
        contrib = jnp.dot(oh, aug, preferred_element_type=jnp.float32)

        @pl.when(i == 0)
        def _():
            acc_ref[...] = jnp.zeros_like(acc_ref)
        acc_ref[...] += contrib

        @pl.when(i == nsteps - 1)
        def _():
            acc = acc_ref[...]
            seg = acc[:, :HV]
            cnt = acc[:, HV][:, None]
            x2 = seg / jnp.maximum(cnt, 1.0)
            hfc = jnp.maximum(
                jnp.dot(x2, w1_ref[...], preferred_element_type=jnp.float32)
                + c1_ref[...], 0.0)
            logits = jnp.dot(hfc, w2_ref[...],
                             preferred_element_type=jnp.float32) + c2_ref[...]
            col = jax.lax.broadcasted_iota(jnp.int32, (G, 128), 1)
            valid = col < 2
            neg = jnp.full_like(logits, -jnp.inf)
            m = jnp.max(jnp.where(valid, logits, neg), axis=1, keepdims=True)
            e = jnp.where(valid, jnp.exp(logits - m), 0.0)
            lse = jnp.log(jnp.sum(e, axis=1, keepdims=True))
            res = logits - m - lse
            o_ref[...] = res[:, :2]

    return pl.pallas_call(
        body,
        grid=grid,
        in_specs=[
            pl.BlockSpec((NC, R, H2), lambda i: (0, i, 0)),
            pl.BlockSpec((R, H2), lambda i: (i, 0)),
            pl.BlockSpec((R, 1), lambda i: (i, 0)),
            pl.BlockSpec((1, H2), lambda i: (0, 0)),
            pl.BlockSpec((R, 1), lambda i: (i, 0)),
            pl.BlockSpec((HV, FH), lambda i: (0, 0)),
            pl.BlockSpec((1, FH), lambda i: (0, 0)),
            pl.BlockSpec((FH, 128), lambda i: (0, 0)),
            pl.BlockSpec((1, 128), lambda i: (0, 0)),
        ],
        out_specs=pl.BlockSpec((G, 2), lambda i: (0, 0)),
        out_shape=jax.ShapeDtypeStruct((G, 2), jnp.float32),
        scratch_shapes=[pltpu.VMEM((G, 128), jnp.float32)],
    )(agg2, h2p, dis, b2, batch, fW1, fb1, fW2p, fb2p)


def kernel(x, edge_index, edge_attr, batch, W1, b1, W2, b2,
           fW1, fb1, fW2, fb2):
    N, D = x.shape
    H1 = W1.shape[1]
    H2 = W2.shape[1]
    E = edge_index.shape[1]

    src = edge_index[0]
    dst = edge_index[1]
    ew = edge_attr[:, 0]

    per = NW * CH * 2
    Epad = (E + per - 1) // per * per
    pad = Epad - E
    if pad:
        zi = jnp.zeros((pad,), jnp.int32)
        src = jnp.concatenate([src, zi])
        dst = jnp.concatenate([dst, zi])
        ew = jnp.concatenate([ew, jnp.zeros((pad,), jnp.float32)])
    cpt = Epad // (NW * CH)
    nchunks = Epad // CH
    eidx = jnp.stack([src, dst], axis=0)
    eidx = eidx.reshape(2, nchunks, CH).transpose(1, 0, 2)
    eww = ew.reshape(nchunks, CH)

    R = 1000
    xw = _stage_a1(x, W1, R)
    degp = _make_deg(N, cpt)(eidx, eww).reshape(NC, N)
    degp_t = degp.T
    h1p, dis = _stage_a2(degp_t, xw, R)
    agg1 = _make_agg(N, H1, cpt)(eidx, eww, h1p)
    W2p = jnp.pad(W2, ((0, 0), (0, 128 - H2)))
    h2p = _stage_b(agg1, h1p, dis, b1.reshape(1, -1), W2p, R)
    agg2 = _make_agg(N, 128, cpt)(eidx, eww, h2p)

    b2p = jnp.pad(b2, (0, 128 - H2)).reshape(1, -1)
    fW2p = jnp.pad(fW2, ((0, 0), (0, 128 - fW2.shape[1])))
    fb2p = jnp.pad(fb2, (0, 128 - fb2.shape[0])).reshape(1, -1)
    out = _stage_c(agg2, h2p, dis, b2p, batch.reshape(-1, 1),
                   fW1, fb1.reshape(1, -1), fW2p, fb2p, H2, R)
    return out
